# baseline (speedup 1.0000x reference)
import functools

import jax
import jax.numpy as jnp
from jax import lax
from jax.experimental import pallas as pl
from jax.experimental.pallas import tpu as pltpu
from jax.experimental.pallas import tpu_sc as plsc

E = 8
K = 2
D = 1024
INTER = 512
T = 2048

BLK = 512
NBLK = (K * T + E * (BLK - 1)) // BLK + 1
NPAD = NBLK * BLK

NW = 32
TPW = T // NW

CUMBLK = 1024


def _pack_bf16(v):
    lo = lax.bitcast_convert_type(v[:, :D // 2].astype(jnp.bfloat16),
                                  jnp.int16).astype(jnp.int32)
    hi = lax.bitcast_convert_type(v[:, D // 2:].astype(jnp.bfloat16),
                                  jnp.int16).astype(jnp.int32)
    return lax.shift_left(hi, 16) | (lo & 0xFFFF)


def _unpack_bf16(w):
    lo = lax.bitcast_convert_type((w & 0xFFFF).astype(jnp.int16),
                                  jnp.bfloat16).astype(jnp.float32)
    hi = lax.bitcast_convert_type(
        lax.shift_right_logical(w, 16).astype(jnp.int16),
        jnp.bfloat16).astype(jnp.float32)
    return jnp.concatenate([lo, hi], axis=1)


def _gate_body(x_ref, gw_ref, sw1_ref, sb1_ref, sw3_ref, sb3_ref, sw2_ref,
               sb2_ref, w1_ref, w2_ref, p1_ref, p2_ref, be_ref, l_ref,
               xb_ref, z_ref, rank_ref, oh_ref):
    x = x_ref[...]
    xb_ref[...] = _pack_bf16(x)
    sa = lax.dot_general(x, sw1_ref[...], (((1,), (1,)), ((), ())),
                         preferred_element_type=jnp.float32) + sb1_ref[...]
    sc = lax.dot_general(x, sw3_ref[...], (((1,), (1,)), ((), ())),
                         preferred_element_type=jnp.float32) + sb3_ref[...]
    sh = sa * (1.0 / (1.0 + jnp.exp(-sa))) * sc
    z_ref[...] = lax.dot_general(sh, sw2_ref[...], (((1,), (1,)), ((), ())),
                                 preferred_element_type=jnp.float32
                                 ) + sb2_ref[...]
    gw = gw_ref[...]
    s = lax.dot_general(x, gw, (((1,), (1,)), ((), ())),
                        preferred_element_type=jnp.float32)
    sig = 1.0 / (1.0 + jnp.exp(-s))
    iota = lax.broadcasted_iota(jnp.int32, (T, E), 1)
    m1 = jnp.max(sig, axis=1, keepdims=True)
    e1 = jnp.min(jnp.where(sig == m1, iota, E), axis=1, keepdims=True)
    sig_m = jnp.where(iota == e1, -1.0, sig)
    m2 = jnp.max(sig_m, axis=1, keepdims=True)
    e2 = jnp.min(jnp.where(sig_m == m2, iota, E), axis=1, keepdims=True)
    den = m1 + m2
    w1_ref[...] = m1 / den
    w2_ref[...] = m2 / den
    oh1 = (iota == e1).astype(jnp.float32)
    oh2 = (iota == e2).astype(jnp.float32)
    cnt1 = jnp.sum(oh1, axis=0, keepdims=True)
    counts = cnt1 + jnp.sum(oh2, axis=0, keepdims=True)
    probs = jnp.sum((m1 / den) * oh1 + (m2 / den) * oh2, axis=0, keepdims=True)
    l_ref[...] = jnp.sum(E * counts / (K * T) * probs / T,
                         axis=1, keepdims=True)

    oh = jnp.concatenate([oh1, oh2], axis=0)
    oh_ref[...] = oh
    ltri = (lax.broadcasted_iota(jnp.int32, (CUMBLK, CUMBLK), 0) >
            lax.broadcasted_iota(jnp.int32, (CUMBLK, CUMBLK), 1)
            ).astype(jnp.float32)

    def step(j, carry):
        blk = oh_ref[pl.ds(j * CUMBLK, CUMBLK), :]
        c = lax.dot_general(ltri, blk, (((1,), (0,)), ((), ())),
                            preferred_element_type=jnp.float32) + carry
        rank_ref[pl.ds(j * CUMBLK, CUMBLK), :] = c
        return carry + jnp.sum(blk, axis=0, keepdims=True)

    lax.fori_loop(0, (K * T) // CUMBLK, step, jnp.zeros((1, E), jnp.float32))
    rank = rank_ref[...]

    cnt_pad = jnp.ceil(counts / BLK) * BLK
    upper = (lax.broadcasted_iota(jnp.int32, (E, E), 0) <
             lax.broadcasted_iota(jnp.int32, (E, E), 1)).astype(jnp.float32)
    off = lax.dot_general(cnt_pad, upper, (((1,), (0,)), ((), ())),
                          preferred_element_type=jnp.float32)
    pos = jnp.sum((off + rank) * oh, axis=1, keepdims=True)
    pos = pos.astype(jnp.int32)
    p1_ref[...] = pos[:T, :]
    p2_ref[...] = pos[T:, :]

    ends = off + cnt_pad
    bstart = (lax.broadcasted_iota(jnp.int32, (NBLK + 1, E), 0) * BLK
              ).astype(jnp.float32)
    be = jnp.sum((bstart >= ends).astype(jnp.int32), axis=1, keepdims=True)
    nblk_used = jnp.sum(cnt_pad, axis=1, keepdims=True) / BLK
    be_ref[...] = jnp.where(
        lax.broadcasted_iota(jnp.int32, (NBLK + 1, 1), 0) < NBLK,
        jnp.clip(be, 0, E - 1), nblk_used.astype(jnp.int32))


def _gate_call(x, gate_w, sw1, sb1, sw3, sb3, sw2, sb2):
    out_shape = (
        jax.ShapeDtypeStruct((T, 1), jnp.float32),
        jax.ShapeDtypeStruct((T, 1), jnp.float32),
        jax.ShapeDtypeStruct((T, 1), jnp.int32),
        jax.ShapeDtypeStruct((T, 1), jnp.int32),
        jax.ShapeDtypeStruct((NBLK + 1, 1), jnp.int32),
        jax.ShapeDtypeStruct((1, 1), jnp.float32),
        jax.ShapeDtypeStruct((T, D // 2), jnp.int32),
        jax.ShapeDtypeStruct((T, D), jnp.float32),
    )
    return pl.pallas_call(
        _gate_body,
        out_shape=out_shape,
        scratch_shapes=[pltpu.VMEM((K * T, E), jnp.float32),
                        pltpu.VMEM((K * T, E), jnp.float32)],
    )(x, gate_w, sw1, sb1, sw3, sb3, sw2, sb2)


def _mlp_body(be_ref, xs_ref, w1_ref, w3_ref, w2_ref, out_ref):
    @pl.when(pl.program_id(0) < be_ref[NBLK])
    def _():
        xb = _unpack_bf16(xs_ref[...])
        a = lax.dot_general(xb, w1_ref[0], (((1,), (1,)), ((), ())),
                            preferred_element_type=jnp.float32)
        c = lax.dot_general(xb, w3_ref[0], (((1,), (1,)), ((), ())),
                            preferred_element_type=jnp.float32)
        h = a * (1.0 / (1.0 + jnp.exp(-a))) * c
        y = lax.dot_general(h, w2_ref[0], (((1,), (1,)), ((), ())),
                            preferred_element_type=jnp.float32)
        out_ref[...] = _pack_bf16(y)


def _mlp_call(be, xs, W1, W3, W2):
    grid_spec = pltpu.PrefetchScalarGridSpec(
        num_scalar_prefetch=1,
        grid=(NBLK,),
        in_specs=[
            pl.BlockSpec((BLK, D // 2), lambda i, be: (i, 0)),
            pl.BlockSpec((1, INTER, D), lambda i, be: (be[i], 0, 0)),
            pl.BlockSpec((1, INTER, D), lambda i, be: (be[i], 0, 0)),
            pl.BlockSpec((1, D, INTER), lambda i, be: (be[i], 0, 0)),
        ],
        out_specs=pl.BlockSpec((BLK, D // 2), lambda i, be: (i, 0)),
    )
    return pl.pallas_call(
        _mlp_body,
        grid_spec=grid_spec,
        out_shape=jax.ShapeDtypeStruct((NPAD, D // 2), jnp.int32),
    )(be, xs, W1, W3, W2)


def _shared_z_body(x_ref, sw1_ref, sb1_ref, sw3_ref, sb3_ref, sw2_ref,
                   sb2_ref, out_ref):
    xb = x_ref[...]
    a = lax.dot_general(xb, sw1_ref[...], (((1,), (1,)), ((), ())),
                        preferred_element_type=jnp.float32) + sb1_ref[...]
    c = lax.dot_general(xb, sw3_ref[...], (((1,), (1,)), ((), ())),
                        preferred_element_type=jnp.float32) + sb3_ref[...]
    h = a * (1.0 / (1.0 + jnp.exp(-a))) * c
    out_ref[...] = lax.dot_general(h, sw2_ref[...], (((1,), (1,)), ((), ())),
                                   preferred_element_type=jnp.float32
                                   ) + sb2_ref[...]


def _shared_z_call(x, sw1, sb1, sw3, sb3, sw2, sb2):
    tb = 512
    row = lambda i: (i, 0)
    full = lambda i: (0, 0)
    return pl.pallas_call(
        _shared_z_body,
        grid=(T // tb,),
        in_specs=[
            pl.BlockSpec((tb, D), row),
            pl.BlockSpec((INTER, D), full),
            pl.BlockSpec((1, INTER), full),
            pl.BlockSpec((INTER, D), full),
            pl.BlockSpec((1, INTER), full),
            pl.BlockSpec((D, INTER), full),
            pl.BlockSpec((1, D), full),
        ],
        out_specs=pl.BlockSpec((tb, D), row),
        out_shape=jax.ShapeDtypeStruct((T, D), jnp.float32),
    )(x, sw1, sb1, sw3, sb3, sw2, sb2)


def _combine_body(z_ref, g0_ref, g1_ref, w1_ref, w2_ref, out_ref):
    out_ref[...] = (z_ref[...] + w1_ref[...] * _unpack_bf16(g0_ref[...])
                    + w2_ref[...] * _unpack_bf16(g1_ref[...]))


def _combine_call(z, g0, g1, w1, w2):
    tb = 1024
    row = lambda i: (i, 0)
    return pl.pallas_call(
        _combine_body,
        grid=(T // tb,),
        in_specs=[
            pl.BlockSpec((tb, D), row),
            pl.BlockSpec((tb, D // 2), row),
            pl.BlockSpec((tb, D // 2), row),
            pl.BlockSpec((tb, 1), row),
            pl.BlockSpec((tb, 1), row),
        ],
        out_specs=pl.BlockSpec((tb, D), row),
        out_shape=jax.ShapeDtypeStruct((T, D), jnp.float32),
    )(z, g0, g1, w1, w2)


@functools.cache
def _sc_scatter_kernel():
    mesh = plsc.VectorSubcoreMesh(core_axis_name="c", subcore_axis_name="s")

    @functools.partial(
        pl.kernel,
        out_type=jax.ShapeDtypeStruct((NPAD, D // 2), jnp.int32),
        mesh=mesh,
        scratch_types=[
            pltpu.VMEM((TPW,), jnp.int32),
            pltpu.VMEM((TPW,), jnp.int32),
            pltpu.VMEM((TPW, D // 2), jnp.int32),
            pltpu.SemaphoreType.DMA,
            pltpu.SemaphoreType.DMA,
        ],
    )
    def _sc_scatter(x_hbm, p1_hbm, p2_hbm, xs_hbm, idx1_v, idx2_v, rows_v,
                    sem1, sem2):
        wid = lax.axis_index("s") * 2 + lax.axis_index("c")
        base = wid * TPW
        pltpu.sync_copy(p1_hbm.at[pl.ds(base, TPW)], idx1_v)
        pltpu.sync_copy(p2_hbm.at[pl.ds(base, TPW)], idx2_v)
        pltpu.sync_copy(x_hbm.at[pl.ds(base, TPW)], rows_v)
        cp1 = pltpu.async_copy(rows_v, xs_hbm.at[idx1_v], sem1)
        cp2 = pltpu.async_copy(rows_v, xs_hbm.at[idx2_v], sem2)
        cp1.wait()
        cp2.wait()

    return _sc_scatter


@functools.cache
def _sc_gather_kernel():
    mesh = plsc.VectorSubcoreMesh(core_axis_name="c", subcore_axis_name="s")

    @functools.partial(
        pl.kernel,
        out_type=(jax.ShapeDtypeStruct((T, D // 2), jnp.int32),
                  jax.ShapeDtypeStruct((T, D // 2), jnp.int32)),
        mesh=mesh,
        scratch_types=[
            pltpu.VMEM((TPW,), jnp.int32),
            pltpu.VMEM((TPW,), jnp.int32),
            pltpu.VMEM((TPW, D // 2), jnp.int32),
            pltpu.VMEM((TPW, D // 2), jnp.int32),
            pltpu.SemaphoreType.DMA,
            pltpu.SemaphoreType.DMA,
        ],
    )
    def _sc_gather(ys_hbm, p1_hbm, p2_hbm, g0_hbm, g1_hbm, idx1_v, idx2_v,
                   rows1_v, rows2_v, sem1, sem2):
        wid = lax.axis_index("s") * 2 + lax.axis_index("c")
        base = wid * TPW
        pltpu.sync_copy(p1_hbm.at[pl.ds(base, TPW)], idx1_v)
        pltpu.sync_copy(p2_hbm.at[pl.ds(base, TPW)], idx2_v)
        cp1 = pltpu.async_copy(ys_hbm.at[idx1_v], rows1_v, sem1)
        cp2 = pltpu.async_copy(ys_hbm.at[idx2_v], rows2_v, sem2)
        cp1.wait()
        pltpu.sync_copy(rows1_v, g0_hbm.at[pl.ds(base, TPW)])
        cp2.wait()
        pltpu.sync_copy(rows2_v, g1_hbm.at[pl.ds(base, TPW)])

    return _sc_gather


def kernel(x, gate_w, W1, W2, W3, sw1, sb1, sw2, sb2, sw3, sb3):
    w1, w2, pos1, pos2, be, aux, xb, z = _gate_call(
        x, gate_w, sw1, sb1.reshape(1, INTER), sw3, sb3.reshape(1, INTER),
        sw2, sb2.reshape(1, D))
    p1 = pos1.reshape(T)
    p2 = pos2.reshape(T)
    xs = _sc_scatter_kernel()(xb, p1, p2)
    ys = _mlp_call(be.reshape(NBLK + 1), xs, W1, W3, W2)
    g0, g1 = _sc_gather_kernel()(ys, p1, p2)
    y = _combine_call(z, g0, g1, w1, w2)
    return (y, aux.reshape(()))


# speedup vs baseline: 1.0586x; 1.0586x over previous
import functools

import jax
import jax.numpy as jnp
from jax import lax
from jax.experimental import pallas as pl
from jax.experimental.pallas import tpu as pltpu
from jax.experimental.pallas import tpu_sc as plsc

E = 8
K = 2
D = 1024
INTER = 512
T = 2048

BLK = 512
NBLK = (K * T + E * (BLK - 1)) // BLK + 1
NPAD = NBLK * BLK

NW = 32
TPW = T // NW

CUMBLK = 1024


def _pack_bf16(v):
    lo = lax.bitcast_convert_type(v[:, :D // 2].astype(jnp.bfloat16),
                                  jnp.int16).astype(jnp.int32)
    hi = lax.bitcast_convert_type(v[:, D // 2:].astype(jnp.bfloat16),
                                  jnp.int16).astype(jnp.int32)
    return lax.shift_left(hi, 16) | (lo & 0xFFFF)


def _unpack_bf16(w):
    lo = lax.bitcast_convert_type((w & 0xFFFF).astype(jnp.int16),
                                  jnp.bfloat16).astype(jnp.float32)
    hi = lax.bitcast_convert_type(
        lax.shift_right_logical(w, 16).astype(jnp.int16),
        jnp.bfloat16).astype(jnp.float32)
    return jnp.concatenate([lo, hi], axis=1)


def _gate_body(x_ref, gw_ref, w1_ref, w2_ref, p1_ref, p2_ref, be_ref, l_ref,
               xb_ref, rank_ref, oh_ref):
    x = x_ref[...]
    xb_ref[...] = _pack_bf16(x)
    gw = gw_ref[...]
    s = lax.dot_general(x, gw, (((1,), (1,)), ((), ())),
                        preferred_element_type=jnp.float32)
    sig = 1.0 / (1.0 + jnp.exp(-s))
    iota = lax.broadcasted_iota(jnp.int32, (T, E), 1)
    m1 = jnp.max(sig, axis=1, keepdims=True)
    e1 = jnp.min(jnp.where(sig == m1, iota, E), axis=1, keepdims=True)
    sig_m = jnp.where(iota == e1, -1.0, sig)
    m2 = jnp.max(sig_m, axis=1, keepdims=True)
    e2 = jnp.min(jnp.where(sig_m == m2, iota, E), axis=1, keepdims=True)
    den = m1 + m2
    w1_ref[...] = m1 / den
    w2_ref[...] = m2 / den
    oh1 = (iota == e1).astype(jnp.float32)
    oh2 = (iota == e2).astype(jnp.float32)
    cnt1 = jnp.sum(oh1, axis=0, keepdims=True)
    counts = cnt1 + jnp.sum(oh2, axis=0, keepdims=True)
    probs = jnp.sum((m1 / den) * oh1 + (m2 / den) * oh2, axis=0, keepdims=True)
    l_ref[...] = jnp.sum(E * counts / (K * T) * probs / T,
                         axis=1, keepdims=True)

    oh = jnp.concatenate([oh1, oh2], axis=0)
    oh_ref[...] = oh
    ltri = (lax.broadcasted_iota(jnp.int32, (CUMBLK, CUMBLK), 0) >
            lax.broadcasted_iota(jnp.int32, (CUMBLK, CUMBLK), 1)
            ).astype(jnp.float32)

    def step(j, carry):
        blk = oh_ref[pl.ds(j * CUMBLK, CUMBLK), :]
        c = lax.dot_general(ltri, blk, (((1,), (0,)), ((), ())),
                            preferred_element_type=jnp.float32) + carry
        rank_ref[pl.ds(j * CUMBLK, CUMBLK), :] = c
        return carry + jnp.sum(blk, axis=0, keepdims=True)

    lax.fori_loop(0, (K * T) // CUMBLK, step, jnp.zeros((1, E), jnp.float32))
    rank = rank_ref[...]

    cnt_pad = jnp.ceil(counts / BLK) * BLK
    upper = (lax.broadcasted_iota(jnp.int32, (E, E), 0) <
             lax.broadcasted_iota(jnp.int32, (E, E), 1)).astype(jnp.float32)
    off = lax.dot_general(cnt_pad, upper, (((1,), (0,)), ((), ())),
                          preferred_element_type=jnp.float32)
    pos = jnp.sum((off + rank) * oh, axis=1, keepdims=True)
    pos = pos.astype(jnp.int32)
    p1_ref[...] = pos[:T, :]
    p2_ref[...] = pos[T:, :]

    ends = off + cnt_pad
    bstart = (lax.broadcasted_iota(jnp.int32, (NBLK + 1, E), 0) * BLK
              ).astype(jnp.float32)
    be = jnp.sum((bstart >= ends).astype(jnp.int32), axis=1, keepdims=True)
    nblk_used = jnp.sum(cnt_pad, axis=1, keepdims=True) / BLK
    be_ref[...] = jnp.where(
        lax.broadcasted_iota(jnp.int32, (NBLK + 1, 1), 0) < NBLK,
        jnp.clip(be, 0, E - 1), nblk_used.astype(jnp.int32))


def _gate_call(x, gate_w):
    out_shape = (
        jax.ShapeDtypeStruct((T, 1), jnp.float32),
        jax.ShapeDtypeStruct((T, 1), jnp.float32),
        jax.ShapeDtypeStruct((T, 1), jnp.int32),
        jax.ShapeDtypeStruct((T, 1), jnp.int32),
        jax.ShapeDtypeStruct((NBLK + 1, 1), jnp.int32),
        jax.ShapeDtypeStruct((1, 1), jnp.float32),
        jax.ShapeDtypeStruct((T, D // 2), jnp.int32),
    )
    return pl.pallas_call(
        _gate_body,
        out_shape=out_shape,
        scratch_shapes=[pltpu.VMEM((K * T, E), jnp.float32),
                        pltpu.VMEM((K * T, E), jnp.float32)],
    )(x, gate_w)


def _mlp_body(be_ref, xs_ref, w1_ref, w3_ref, w2_ref, out_ref):
    @pl.when(pl.program_id(0) < be_ref[NBLK])
    def _():
        xb = _unpack_bf16(xs_ref[...])
        a = lax.dot_general(xb, w1_ref[0], (((1,), (1,)), ((), ())),
                            preferred_element_type=jnp.float32)
        c = lax.dot_general(xb, w3_ref[0], (((1,), (1,)), ((), ())),
                            preferred_element_type=jnp.float32)
        h = a * (1.0 / (1.0 + jnp.exp(-a))) * c
        y = lax.dot_general(h, w2_ref[0], (((1,), (1,)), ((), ())),
                            preferred_element_type=jnp.float32)
        out_ref[...] = _pack_bf16(y)


def _mlp_call(be, xs, W1, W3, W2):
    grid_spec = pltpu.PrefetchScalarGridSpec(
        num_scalar_prefetch=1,
        grid=(NBLK,),
        in_specs=[
            pl.BlockSpec((BLK, D // 2), lambda i, be: (i, 0)),
            pl.BlockSpec((1, INTER, D), lambda i, be: (be[i], 0, 0)),
            pl.BlockSpec((1, INTER, D), lambda i, be: (be[i], 0, 0)),
            pl.BlockSpec((1, D, INTER), lambda i, be: (be[i], 0, 0)),
        ],
        out_specs=pl.BlockSpec((BLK, D // 2), lambda i, be: (i, 0)),
    )
    return pl.pallas_call(
        _mlp_body,
        grid_spec=grid_spec,
        out_shape=jax.ShapeDtypeStruct((NPAD, D // 2), jnp.int32),
    )(be, xs, W1, W3, W2)


def _shared_z_body(x_ref, sw1_ref, sb1_ref, sw3_ref, sb3_ref, sw2_ref,
                   sb2_ref, out_ref):
    xb = x_ref[...]
    a = lax.dot_general(xb, sw1_ref[...], (((1,), (1,)), ((), ())),
                        preferred_element_type=jnp.float32) + sb1_ref[...]
    c = lax.dot_general(xb, sw3_ref[...], (((1,), (1,)), ((), ())),
                        preferred_element_type=jnp.float32) + sb3_ref[...]
    h = a * (1.0 / (1.0 + jnp.exp(-a))) * c
    out_ref[...] = lax.dot_general(h, sw2_ref[...], (((1,), (1,)), ((), ())),
                                   preferred_element_type=jnp.float32
                                   ) + sb2_ref[...]


def _shared_z_call(x, sw1, sb1, sw3, sb3, sw2, sb2):
    tb = 512
    row = lambda i: (i, 0)
    full = lambda i: (0, 0)
    return pl.pallas_call(
        _shared_z_body,
        grid=(T // tb,),
        in_specs=[
            pl.BlockSpec((tb, D), row),
            pl.BlockSpec((INTER, D), full),
            pl.BlockSpec((1, INTER), full),
            pl.BlockSpec((INTER, D), full),
            pl.BlockSpec((1, INTER), full),
            pl.BlockSpec((D, INTER), full),
            pl.BlockSpec((1, D), full),
        ],
        out_specs=pl.BlockSpec((tb, D), row),
        out_shape=jax.ShapeDtypeStruct((T, D), jnp.float32),
    )(x, sw1, sb1, sw3, sb3, sw2, sb2)


def _combine_body(z_ref, g0_ref, g1_ref, w1_ref, w2_ref, out_ref):
    out_ref[...] = (z_ref[...] + w1_ref[...] * _unpack_bf16(g0_ref[...])
                    + w2_ref[...] * _unpack_bf16(g1_ref[...]))


def _combine_call(z, g0, g1, w1, w2):
    tb = 1024
    row = lambda i: (i, 0)
    return pl.pallas_call(
        _combine_body,
        grid=(T // tb,),
        in_specs=[
            pl.BlockSpec((tb, D), row),
            pl.BlockSpec((tb, D // 2), row),
            pl.BlockSpec((tb, D // 2), row),
            pl.BlockSpec((tb, 1), row),
            pl.BlockSpec((tb, 1), row),
        ],
        out_specs=pl.BlockSpec((tb, D), row),
        out_shape=jax.ShapeDtypeStruct((T, D), jnp.float32),
    )(z, g0, g1, w1, w2)


@functools.cache
def _sc_scatter_kernel():
    mesh = plsc.VectorSubcoreMesh(core_axis_name="c", subcore_axis_name="s")

    @functools.partial(
        pl.kernel,
        out_type=jax.ShapeDtypeStruct((NPAD, D // 2), jnp.int32),
        mesh=mesh,
        scratch_types=[
            pltpu.VMEM((TPW,), jnp.int32),
            pltpu.VMEM((TPW,), jnp.int32),
            pltpu.VMEM((TPW, D // 2), jnp.int32),
            pltpu.SemaphoreType.DMA,
            pltpu.SemaphoreType.DMA,
        ],
    )
    def _sc_scatter(x_hbm, p1_hbm, p2_hbm, xs_hbm, idx1_v, idx2_v, rows_v,
                    sem1, sem2):
        wid = lax.axis_index("s") * 2 + lax.axis_index("c")
        base = wid * TPW
        pltpu.sync_copy(p1_hbm.at[pl.ds(base, TPW)], idx1_v)
        pltpu.sync_copy(p2_hbm.at[pl.ds(base, TPW)], idx2_v)
        pltpu.sync_copy(x_hbm.at[pl.ds(base, TPW)], rows_v)
        cp1 = pltpu.async_copy(rows_v, xs_hbm.at[idx1_v], sem1)
        cp2 = pltpu.async_copy(rows_v, xs_hbm.at[idx2_v], sem2)
        cp1.wait()
        cp2.wait()

    return _sc_scatter


@functools.cache
def _sc_gather_kernel():
    mesh = plsc.VectorSubcoreMesh(core_axis_name="c", subcore_axis_name="s")

    @functools.partial(
        pl.kernel,
        out_type=(jax.ShapeDtypeStruct((T, D // 2), jnp.int32),
                  jax.ShapeDtypeStruct((T, D // 2), jnp.int32)),
        mesh=mesh,
        scratch_types=[
            pltpu.VMEM((TPW,), jnp.int32),
            pltpu.VMEM((TPW,), jnp.int32),
            pltpu.VMEM((TPW, D // 2), jnp.int32),
            pltpu.VMEM((TPW, D // 2), jnp.int32),
            pltpu.SemaphoreType.DMA,
            pltpu.SemaphoreType.DMA,
        ],
    )
    def _sc_gather(ys_hbm, p1_hbm, p2_hbm, g0_hbm, g1_hbm, idx1_v, idx2_v,
                   rows1_v, rows2_v, sem1, sem2):
        wid = lax.axis_index("s") * 2 + lax.axis_index("c")
        base = wid * TPW
        pltpu.sync_copy(p1_hbm.at[pl.ds(base, TPW)], idx1_v)
        pltpu.sync_copy(p2_hbm.at[pl.ds(base, TPW)], idx2_v)
        cp1 = pltpu.async_copy(ys_hbm.at[idx1_v], rows1_v, sem1)
        cp2 = pltpu.async_copy(ys_hbm.at[idx2_v], rows2_v, sem2)
        cp1.wait()
        pltpu.sync_copy(rows1_v, g0_hbm.at[pl.ds(base, TPW)])
        cp2.wait()
        pltpu.sync_copy(rows2_v, g1_hbm.at[pl.ds(base, TPW)])

    return _sc_gather


def kernel(x, gate_w, W1, W2, W3, sw1, sb1, sw2, sb2, sw3, sb3):
    w1, w2, pos1, pos2, be, aux, xb = _gate_call(x, gate_w)
    p1 = pos1.reshape(T)
    p2 = pos2.reshape(T)
    z = _shared_z_call(x, sw1, sb1.reshape(1, INTER), sw3,
                       sb3.reshape(1, INTER), sw2, sb2.reshape(1, D))
    xs = _sc_scatter_kernel()(xb, p1, p2)
    ys = _mlp_call(be.reshape(NBLK + 1), xs, W1, W3, W2)
    g0, g1 = _sc_gather_kernel()(ys, p1, p2)
    y = _combine_call(z, g0, g1, w1, w2)
    return (y, aux.reshape(()))
